# Initial kernel scaffold; baseline (speedup 1.0000x reference)
import jax
import jax.numpy as jnp
from jax.experimental import pallas as pl


def kernel(observations, edge_index, W_enc1, b_enc1, W_enc2, b_enc2, W_g1, att_src1, att_dst1, b_g1, W_g2, att_src2, att_dst2, b_g2, W_dec1, b_dec1, W_dec2, b_dec2):
    raise NotImplementedError("write your pallas kernel here")


# speedup vs baseline: 21.7090x; 21.7090x over previous
import jax
import jax.numpy as jnp
from jax import lax
from jax.experimental import pallas as pl
from jax.experimental.pallas import tpu as pltpu
from jax.experimental.pallas import tpu_sc as plsc

N = 50000
E = 800000
OBS = 128
HID = 64
H = 4
ACT = 32
F = H * HID
FW = 384
AW = 128
NW = 32
ESUB = E // NW
BK = 128
NB = ESUB // BK + 1
WACC = 128
NU = 50176

_f32 = jnp.float32
_i32 = jnp.int32


def _sc_gat(srcs, dsts, hplus, a8p, dfirst_hbm,
            u_hbm, bnd_hbm,
            acc, rows, a8d, wbuf, sidx, dbuf, sidh, dbh, dfv,
            semh, sema):
    cid = lax.axis_index("c")
    sid = lax.axis_index("s")
    wid = cid * 16 + sid
    ebase = wid * ESUB
    ii = lax.iota(_i32, 16)
    z16f = jnp.zeros((16,), _f32)

    def _zacc(r, c):
        for v in range(FW // 16):
            acc[r, pl.ds(v * 16, 16)] = z16f
        return c
    lax.fori_loop(0, WACC, _zacc, 0)

    def _zw(r, c):
        wbuf[r, pl.ds(0, 16)] = z16f
        return c
    lax.fori_loop(0, 64, _zw, 0)

    pltpu.sync_copy(dfirst_hbm, dfv)

    ubase = cid * NU
    def _zu(b, c):
        pltpu.sync_copy(acc.at[pl.ds(0, 8)],
                        u_hbm.at[pl.ds(ubase + sid * 3136 + b * 8, 8)])
        return c
    lax.fori_loop(0, 392, _zu, 0)
    plsc.subcore_barrier()

    def _flush_row(r, fdst):
        phys = lax.rem(r, WACC)

        @pl.when(r == fdst)
        def _():
            pltpu.sync_copy(acc.at[pl.ds(phys, 1)],
                            bnd_hbm.at[pl.ds(wid, 1)])

        @pl.when(r != fdst)
        def _():
            pltpu.sync_copy(acc.at[pl.ds(phys, 1)],
                            u_hbm.at[pl.ds(cid * NU + r, 1)])
        for v in range(FW // 16):
            acc[phys, pl.ds(v * 16, 16)] = z16f

    def _batch(b, carry):
        wbase, fdst, dlast = carry
        off = ebase + b * BK
        pltpu.sync_copy(srcs.at[pl.ds(off, BK)], sidx)
        pltpu.sync_copy(dsts.at[pl.ds(off, BK)], dbuf.at[pl.ds(0, BK)])
        nvalid = jnp.minimum(ESUB - b * BK, BK)
        d_first = dbuf[pl.ds(0, 16)][0]
        wbase = jnp.where(wbase < 0, d_first, wbase)
        fdst = jnp.where(fdst < 0, d_first, fdst)

        def _mask(v, dl):
            gl = v * 16 + ii
            d16 = dbuf[pl.ds(v * 16, 16)]
            d16 = jnp.where(gl >= nvalid, d_first, d16)
            dbuf[pl.ds(v * 16, 16)] = d16
            return jnp.maximum(dl, jnp.max(d16))
        dlast = lax.fori_loop(0, 8, _mask, dlast)

        def _half(half, wb0):
            hofs = half * 64

            def _cpidx(v, c):
                sidh[pl.ds(v * 16, 16)] = sidx[pl.ds(hofs + v * 16, 16)]
                dbh[pl.ds(v * 16, 16)] = dbuf[pl.ds(hofs + v * 16, 16)]
                return c
            lax.fori_loop(0, 4, _cpidx, 0)
            cp1 = pltpu.async_copy(hplus.at[sidh], rows, semh)
            cp2 = pltpu.async_copy(a8p.at[dbh], a8d, sema)
            cp1.wait()
            cp2.wait()

            def _wc(kb, c):
                ridx = kb * 16 + ii
                gl = hofs + kb * 16 + ii
                for j in range(H):
                    s = (plsc.load_gather(
                            rows, [ridx, jnp.full((16,), F + j, _i32)])
                         + plsc.load_gather(
                            a8d, [ridx, jnp.full((16,), H + j, _i32)]))
                    w16 = jnp.exp(jnp.where(s > 0, s, 0.2 * s))
                    w16 = jnp.where(gl >= nvalid, 0.0, w16)
                    plsc.store_scatter(
                        wbuf, [ridx, jnp.full((16,), j, _i32)], w16)
                return c
            lax.fori_loop(0, 4, _wc, 0)

            def _grp(g, wb):
                k0 = g * 4
                d16 = dbuf[pl.ds(hofs + k0, 16)]
                for l in range(4):
                    k = k0 + l
                    dk = d16[l]

                    def _adv(w0, _dk=dk):
                        def _fl(r, c):
                            _flush_row(r, fdst)
                            return c
                        lax.fori_loop(w0, _dk, _fl, 0)
                        return _dk
                    wb = lax.cond(dk - wb >= WACC, _adv, lambda w0: w0, wb)
                    phys = lax.rem(dk, WACC)
                    wrow = wbuf[k, pl.ds(0, 16)]
                    for j in range(H):
                        wj = jnp.broadcast_to(wrow[j], (16,))
                        for v in range(4):
                            sl = pl.ds(j * 64 + v * 16, 16)
                            acc[phys, sl] = acc[phys, sl] + wj * rows[k, sl]
                    sl = pl.ds(F, 16)
                    acc[phys, sl] = acc[phys, sl] + wrow
                return wb
            return lax.fori_loop(0, 16, _grp, wb0)
        wbase = lax.fori_loop(0, 2, _half, wbase)
        return (wbase, fdst, dlast)

    wbase, fdst, dlast = lax.fori_loop(
        0, NB, _batch, (jnp.int32(-1), jnp.int32(-1), jnp.int32(-1)))

    def _ffl(r, c):
        _flush_row(r, fdst)
        return c
    lax.fori_loop(wbase, dlast + 1, _ffl, 0)
    plsc.subcore_barrier()

    @pl.when(sid == 0)
    def _():
        def _fix(w, c):
            wc = cid * 16 + w
            dfw = dfv[wc, pl.ds(0, 16)][0]
            pltpu.sync_copy(bnd_hbm.at[pl.ds(wc, 1)], rows.at[pl.ds(0, 1)])
            pltpu.sync_copy(u_hbm.at[pl.ds(ubase + dfw, 1)],
                            rows.at[pl.ds(1, 1)])
            for v in range(FW // 16):
                sl = pl.ds(v * 16, 16)
                rows[1, sl] = rows[1, sl] + rows[0, sl]
            pltpu.sync_copy(rows.at[pl.ds(1, 1)],
                            u_hbm.at[pl.ds(ubase + dfw, 1)])
            return c
        lax.fori_loop(0, 16, _fix, 0)
    plsc.subcore_barrier()


def _sc_gat_call(srcs, dsts, hplus, a8p, dfirst):
    mesh = plsc.VectorSubcoreMesh(core_axis_name="c", subcore_axis_name="s")
    return pl.kernel(
        _sc_gat,
        out_type=(jax.ShapeDtypeStruct((2 * NU, FW), _f32),
                  jax.ShapeDtypeStruct((NW, FW), _f32)),
        mesh=mesh,
        compiler_params=pltpu.CompilerParams(needs_layout_passes=False),
        scratch_types=[
            pltpu.VMEM((WACC, FW), _f32),
            pltpu.VMEM((64, FW), _f32),
            pltpu.VMEM((64, AW), _f32),
            pltpu.VMEM((64, 16), _f32),
            pltpu.VMEM((BK,), _i32),
            pltpu.VMEM((BK + 16,), _i32),
            pltpu.VMEM((64,), _i32),
            pltpu.VMEM((64,), _i32),
            pltpu.VMEM((32, 16), _i32),
            pltpu.SemaphoreType.DMA,
            pltpu.SemaphoreType.DMA,
        ],
    )(srcs, dsts, hplus, a8p, dfirst)


_R = 1000


def _proj_tail(x, wg, a):
    h = jnp.dot(x, wg[...], preferred_element_type=_f32)
    a8 = jnp.dot(h, a[...], preferred_element_type=_f32)
    s = a8[:, :H] + a8[:, H:]
    ws = jnp.exp(jnp.where(s > 0, s, 0.2 * s))
    z = jnp.zeros((x.shape[0], FW - F - 2 * H), _f32)
    hplus = jnp.concatenate([h, a8, z], axis=1)
    a8p = jnp.concatenate([a8, jnp.zeros((x.shape[0], AW - 2 * H), _f32)],
                          axis=1)
    return hplus, a8p, ws


def _enc_body(obs, w1, b1, w2, b2, wg, a, hp_o, a8p_o, ws_o):
    x = jnp.maximum(jnp.dot(obs[...], w1[...],
                            preferred_element_type=_f32) + b1[...], 0.0)
    x = jnp.dot(x, w2[...], preferred_element_type=_f32) + b2[...]
    hp_o[...], a8p_o[...], ws_o[...] = _proj_tail(x, wg, a)


def _enc_call(obs, w1, b1, w2, b2, wg, a):
    return pl.pallas_call(
        _enc_body,
        grid=(N // _R,),
        in_specs=[
            pl.BlockSpec((_R, OBS), lambda i: (i, 0)),
            pl.BlockSpec((OBS, HID), lambda i: (0, 0)),
            pl.BlockSpec((1, HID), lambda i: (0, 0)),
            pl.BlockSpec((HID, HID), lambda i: (0, 0)),
            pl.BlockSpec((1, HID), lambda i: (0, 0)),
            pl.BlockSpec((HID, F), lambda i: (0, 0)),
            pl.BlockSpec((F, 2 * H), lambda i: (0, 0)),
        ],
        out_specs=[
            pl.BlockSpec((_R, FW), lambda i: (i, 0)),
            pl.BlockSpec((_R, AW), lambda i: (i, 0)),
            pl.BlockSpec((_R, H), lambda i: (i, 0)),
        ],
        out_shape=[
            jax.ShapeDtypeStruct((N, FW), _f32),
            jax.ShapeDtypeStruct((N, AW), _f32),
            jax.ShapeDtypeStruct((N, H), _f32),
        ],
    )(obs, w1, b1, w2, b2, wg, a)


def _gat_combine(u0, u1, hp, ws, bg):
    acc = None
    for j in range(H):
        num = (u0[:, j * 64:(j + 1) * 64] + u1[:, j * 64:(j + 1) * 64]
               + ws[:, j:j + 1] * hp[:, j * 64:(j + 1) * 64])
        d = (u0[:, F + j:F + j + 1] + u1[:, F + j:F + j + 1]
             + ws[:, j:j + 1] + 1e-16)
        t = num / d
        acc = t if acc is None else acc + t
    return jnp.maximum(acc * (1.0 / H) + bg, 0.0)


def _comb_body(u0, u1, hp, ws, bg, wg, a, hp_o, a8p_o, ws_o):
    x = _gat_combine(u0[...], u1[...], hp[...], ws[...], bg[...])
    hp_o[...], a8p_o[...], ws_o[...] = _proj_tail(x, wg, a)


def _comb_call(u0, u1, hp, ws, bg, wg, a):
    return pl.pallas_call(
        _comb_body,
        grid=(N // _R,),
        in_specs=[
            pl.BlockSpec((_R, FW), lambda i: (i, 0)),
            pl.BlockSpec((_R, FW), lambda i: (i, 0)),
            pl.BlockSpec((_R, FW), lambda i: (i, 0)),
            pl.BlockSpec((_R, H), lambda i: (i, 0)),
            pl.BlockSpec((1, HID), lambda i: (0, 0)),
            pl.BlockSpec((HID, F), lambda i: (0, 0)),
            pl.BlockSpec((F, 2 * H), lambda i: (0, 0)),
        ],
        out_specs=[
            pl.BlockSpec((_R, FW), lambda i: (i, 0)),
            pl.BlockSpec((_R, AW), lambda i: (i, 0)),
            pl.BlockSpec((_R, H), lambda i: (i, 0)),
        ],
        out_shape=[
            jax.ShapeDtypeStruct((N, FW), _f32),
            jax.ShapeDtypeStruct((N, AW), _f32),
            jax.ShapeDtypeStruct((N, H), _f32),
        ],
    )(u0, u1, hp, ws, bg, wg, a)


def _dec_body(u0, u1, hp, ws, bg, wd1, bd1, wd2, bd2, q_o):
    x = _gat_combine(u0[...], u1[...], hp[...], ws[...], bg[...])
    y = jnp.maximum(jnp.dot(x, wd1[...], preferred_element_type=_f32)
                    + bd1[...], 0.0)
    q_o[...] = jnp.dot(y, wd2[...], preferred_element_type=_f32) + bd2[...]


def _dec_call(u0, u1, hp, ws, bg, wd1, bd1, wd2, bd2):
    return pl.pallas_call(
        _dec_body,
        grid=(N // _R,),
        in_specs=[
            pl.BlockSpec((_R, FW), lambda i: (i, 0)),
            pl.BlockSpec((_R, FW), lambda i: (i, 0)),
            pl.BlockSpec((_R, FW), lambda i: (i, 0)),
            pl.BlockSpec((_R, H), lambda i: (i, 0)),
            pl.BlockSpec((1, HID), lambda i: (0, 0)),
            pl.BlockSpec((HID, HID), lambda i: (0, 0)),
            pl.BlockSpec((1, HID), lambda i: (0, 0)),
            pl.BlockSpec((HID, ACT), lambda i: (0, 0)),
            pl.BlockSpec((1, ACT), lambda i: (0, 0)),
        ],
        out_specs=[pl.BlockSpec((_R, ACT), lambda i: (i, 0))],
        out_shape=[jax.ShapeDtypeStruct((N, ACT), _f32)],
    )(u0, u1, hp, ws, bg, wd1, bd1, wd2, bd2)


def _att_matrix(att_src, att_dst):
    eye = jnp.eye(H, dtype=_f32)
    a_s = jnp.einsum('hc,hk->hck', att_src, eye).reshape(F, H)
    a_d = jnp.einsum('hc,hk->hck', att_dst, eye).reshape(F, H)
    return jnp.concatenate([a_s, a_d], axis=1)


def kernel(observations, edge_index, W_enc1, b_enc1, W_enc2, b_enc2,
           W_g1, att_src1, att_dst1, b_g1, W_g2, att_src2, att_dst2, b_g2,
           W_dec1, b_dec1, W_dec2, b_dec2):
    order = jnp.argsort(edge_index[1])
    src_s = jnp.pad(edge_index[0][order], (0, BK), mode='edge')
    dst_s = jnp.pad(edge_index[1][order], (0, BK), mode='edge')
    dfirst = jnp.broadcast_to(dst_s[0:E:ESUB][:, None], (NW, 16))
    a1 = _att_matrix(att_src1, att_dst1)
    a2 = _att_matrix(att_src2, att_dst2)

    def _layer_u(hp, a8p):
        (u2p, _bnd) = _sc_gat_call(src_s, dst_s, hp, a8p, dfirst)
        return u2p[:N], u2p[NU:NU + N]

    hp1, a8p1, ws1 = _enc_call(observations, W_enc1, b_enc1.reshape(1, HID),
                               W_enc2, b_enc2.reshape(1, HID), W_g1, a1)
    u10, u11 = _layer_u(hp1, a8p1)
    hp2, a8p2, ws2 = _comb_call(u10, u11, hp1, ws1,
                                b_g1.reshape(1, HID), W_g2, a2)
    u20, u21 = _layer_u(hp2, a8p2)
    (q,) = _dec_call(u20, u21, hp2, ws2, b_g2.reshape(1, HID),
                     W_dec1, b_dec1.reshape(1, HID),
                     W_dec2, b_dec2.reshape(1, ACT))
    return q
